# Initial kernel scaffold; baseline (speedup 1.0000x reference)
import jax
import jax.numpy as jnp
from jax.experimental import pallas as pl


def kernel(eigvecs, edge_index, batch_index, g0_W1, g0_b1, g0_W2, g0_b2, g1_W1, g1_b1, g1_W2, g1_b2, g2_W1, g2_b1, g2_W2, g2_b2, rho_W1, rho_b1, rho_W2, rho_b2):
    raise NotImplementedError("write your pallas kernel here")


# speedup vs baseline: 206.6031x; 206.6031x over previous
import jax
import jax.numpy as jnp
from jax import lax
from jax.experimental import pallas as pl
from jax.experimental.pallas import tpu as pltpu
from jax.experimental.pallas import tpu_sc as plsc

N = 50000
NP = 50048
E = 800000
NC = 2
NS = 16
CHUNK = 125
RPB = 4
NPT = NP // NS
TCB = NP // 16

_mesh = plsc.VectorSubcoreMesh(
    core_axis_name="c", subcore_axis_name="s", num_cores=NC, num_subcores=NS)


def _edge_loop(u_ref, src2, dst2, acc, srcb, dstb, rows, sem, idx_row0, nblk):
    def blk(i, carry):
        r = idx_row0 + i * RPB
        pltpu.sync_copy(src2.at[pl.ds(r, RPB)], srcb)
        pltpu.sync_copy(dst2.at[pl.ds(r, RPB)], dstb)
        hs = [pltpu.async_copy(u_ref.at[srcb.at[j]], rows.at[j], sem)
              for j in range(RPB)]
        for h in hs:
            h.wait()
        for j in range(RPB):
            pltpu.sync_copy(rows.at[j], acc.at[dstb.at[j]], add=True)
        return carry

    lax.fori_loop(0, nblk, blk, 0)


def _agg0_body(xaug, zeros16, src2, dst2, p0, p1, acc, srcb, dstb, rows, sem):
    c = lax.axis_index("c")
    s = lax.axis_index("s")
    row0 = s * NPT

    @pl.when(c == 0)
    def _():
        pltpu.sync_copy(xaug.at[pl.ds(row0, NPT)], acc.at[pl.ds(row0, NPT)])

    @pl.when(c == 1)
    def _():
        pltpu.sync_copy(zeros16.at[pl.ds(row0, NPT)], acc.at[pl.ds(row0, NPT)])

    plsc.subcore_barrier()
    w = c * NS + s
    epw = E // (NC * NS)
    _edge_loop(xaug, src2, dst2, acc, srcb, dstb, rows, sem,
               idx_row0=w * (epw // CHUNK), nblk=epw // (CHUNK * RPB))
    plsc.subcore_barrier()

    @pl.when(c == 0)
    def _():
        pltpu.sync_copy(acc.at[pl.ds(row0, NPT)], p0.at[pl.ds(row0, NPT)])

    @pl.when(c == 1)
    def _():
        pltpu.sync_copy(acc.at[pl.ds(row0, NPT)], p1.at[pl.ds(row0, NPT)])


_SC_PARAMS = pltpu.CompilerParams(use_tc_tiling_on_sc=False)

_agg0 = pl.kernel(
    _agg0_body,
    out_type=[jax.ShapeDtypeStruct((NP, 16), jnp.float32),
              jax.ShapeDtypeStruct((NP, 16), jnp.float32)],
    mesh=_mesh,
    compiler_params=_SC_PARAMS,
    scratch_types=[
        pltpu.VMEM_SHARED((NP, 16), jnp.float32),
        pltpu.VMEM((RPB, CHUNK), jnp.int32),
        pltpu.VMEM((RPB, CHUNK), jnp.int32),
        pltpu.VMEM((RPB, CHUNK, 16), jnp.float32),
        pltpu.SemaphoreType.DMA,
    ],
)


def _agg_body(u0, u1, u2, u3, src2, dst2, a0, a1, a2, a3,
              acc, srcb, dstb, rows, sem):
    c = lax.axis_index("c")
    s = lax.axis_index("s")
    row0 = s * NPT
    ept = E // NS

    def do_pass(u_ref, out_ref):
        pltpu.sync_copy(u_ref.at[pl.ds(row0, NPT)], acc.at[pl.ds(row0, NPT)])
        plsc.subcore_barrier()
        _edge_loop(u_ref, src2, dst2, acc, srcb, dstb, rows, sem,
                   idx_row0=s * (ept // CHUNK), nblk=ept // (CHUNK * RPB))
        plsc.subcore_barrier()
        pltpu.sync_copy(acc.at[pl.ds(row0, NPT)], out_ref.at[pl.ds(row0, NPT)])

    @pl.when(c == 0)
    def _():
        do_pass(u0, a0)
        do_pass(u1, a1)

    @pl.when(c == 1)
    def _():
        do_pass(u2, a2)
        do_pass(u3, a3)


_GOUT = [jax.ShapeDtypeStruct((NP, 32), jnp.float32)] * 4
_agg = pl.kernel(
    _agg_body,
    out_type=_GOUT,
    mesh=_mesh,
    compiler_params=_SC_PARAMS,
    scratch_types=[
        pltpu.VMEM_SHARED((NP, 32), jnp.float32),
        pltpu.VMEM((RPB, CHUNK), jnp.int32),
        pltpu.VMEM((RPB, CHUNK), jnp.int32),
        pltpu.VMEM((RPB, CHUNK, 32), jnp.float32),
        pltpu.SemaphoreType.DMA,
    ],
)


def _tc1_body(p0, p1, e0, bd1, b1t, o0, o1, o2, o3):
    m = p0[...] + p1[...]
    a16 = jnp.concatenate([m[:, :8], -m[:, :8]], axis=1)
    z = jax.nn.relu(
        jnp.dot(a16, e0[...], preferred_element_type=jnp.float32) + b1t[...])
    for g, o in enumerate((o0, o1, o2, o3)):
        o[...] = jnp.dot(z[:, g * 32:(g + 1) * 32], bd1[...],
                         preferred_element_type=jnp.float32)


def _tc2_body(a0, a1, a2, a3, p0, p1, bd2, cv1, b1t, o0, o1, o2, o3):
    od = (p0[...] + p1[...])[:, 8:9]
    for g, (a, o) in enumerate(((a0, o0), (a1, o1), (a2, o2), (a3, o3))):
        z = jax.nn.relu(a[...] + od * cv1[:, g * 32:(g + 1) * 32]
                        + b1t[:, g * 32:(g + 1) * 32])
        o[...] = jnp.dot(z, bd2[...], preferred_element_type=jnp.float32)


def _tc3_body(a0, a1, a2, a3, p0, p1, r1, r1b, cv2, b1t, w2, b2, o):
    od = (p0[...] + p1[...])[:, 8:9]
    acc = jnp.broadcast_to(r1b[...], (TCB, 8))
    for g, a in enumerate((a0, a1, a2, a3)):
        z = jax.nn.relu(a[...] + od * cv2[:, g * 32:(g + 1) * 32]
                        + b1t[:, g * 32:(g + 1) * 32])
        acc = acc + jnp.dot(z, r1[g * 32:(g + 1) * 32, :],
                            preferred_element_type=jnp.float32)
    p = jax.nn.relu(acc)
    o[...] = jnp.dot(p, w2[...], preferred_element_type=jnp.float32) + b2[...]


def _row_spec(w):
    return pl.BlockSpec((TCB, w), lambda i: (i, 0))


def _full_spec(shape):
    return pl.BlockSpec(shape, lambda i: tuple(0 for _ in shape))


_GRID = (NP // TCB,)

_tc1 = pl.pallas_call(
    _tc1_body,
    grid=_GRID,
    in_specs=[_row_spec(16), _row_spec(16),
              _full_spec((16, 128)), _full_spec((32, 32)), _full_spec((1, 128))],
    out_specs=[_row_spec(32)] * 4,
    out_shape=_GOUT,
)

_tc2 = pl.pallas_call(
    _tc2_body,
    grid=_GRID,
    in_specs=[_row_spec(32)] * 4 + [_row_spec(16), _row_spec(16),
              _full_spec((32, 32)), _full_spec((1, 128)), _full_spec((1, 128))],
    out_specs=[_row_spec(32)] * 4,
    out_shape=_GOUT,
)

_tc3 = pl.pallas_call(
    _tc3_body,
    grid=_GRID,
    in_specs=[_row_spec(32)] * 4 + [_row_spec(16), _row_spec(16),
              _full_spec((128, 8)), _full_spec((1, 8)), _full_spec((1, 128)),
              _full_spec((1, 128)), _full_spec((8, 8)), _full_spec((1, 8))],
    out_specs=_row_spec(8),
    out_shape=jax.ShapeDtypeStruct((NP, 8), jnp.float32),
)


def kernel(eigvecs, edge_index, batch_index,
           g0_W1, g0_b1, g0_W2, g0_b2,
           g1_W1, g1_b1, g1_W2, g1_b2,
           g2_W1, g2_b1, g2_W2, g2_b2,
           rho_W1, rho_b1, rho_W2, rho_b2):
    f32 = jnp.float32
    x8 = jnp.nan_to_num(eigvecs.astype(f32))
    xaug = jnp.concatenate(
        [x8, jnp.ones((N, 1), f32), jnp.zeros((N, 7), f32)], axis=1)
    xaug = jnp.concatenate([xaug, jnp.zeros((NP - N, 16), f32)], axis=0)
    src2 = edge_index[0].astype(jnp.int32).reshape(E // CHUNK, CHUNK)
    dst2 = edge_index[1].astype(jnp.int32).reshape(E // CHUNK, CHUNK)
    zeros16 = jnp.zeros((NP, 16), f32)

    eye16 = jnp.eye(16, dtype=f32)
    eye4 = jnp.eye(4, dtype=f32)
    e0 = jnp.kron(eye16, g0_W1)
    bd1 = jnp.kron(eye4, g0_W2 @ g1_W1)
    bd2 = jnp.kron(eye4, g1_W2 @ g2_W1)
    b1_0t = jnp.tile(g0_b1, 16)[None]
    cv1 = jnp.tile(g0_b2 @ g1_W1, 16)[None]
    b1_1t = jnp.tile(g1_b1, 16)[None]
    cv2 = jnp.tile(g1_b2 @ g2_W1, 16)[None]
    b1_2t = jnp.tile(g2_b1, 16)[None]
    fold = jnp.kron(jnp.tile(jnp.eye(8, dtype=f32), (2, 1)), g2_W2)
    r1 = fold @ rho_W1
    r1b = (rho_b1 + 2.0 * (jnp.tile(g2_b2, 8) @ rho_W1))[None]
    w2 = rho_W2
    b2 = rho_b2[None]

    p0, p1 = _agg0(xaug, zeros16, src2, dst2)
    u0, u1, u2, u3 = _tc1(p0, p1, e0, bd1, b1_0t)
    A0, A1, A2, A3 = _agg(u0, u1, u2, u3, src2, dst2)
    v0, v1, v2, v3 = _tc2(A0, A1, A2, A3, p0, p1, bd2, cv1, b1_1t)
    B0, B1, B2, B3 = _agg(v0, v1, v2, v3, src2, dst2)
    out = _tc3(B0, B1, B2, B3, p0, p1, r1, r1b, cv2, b1_2t, w2, b2)
    return out[:N]
